# baseline (speedup 1.0000x reference)
import jax
import jax.numpy as jnp
from jax import lax
from jax.experimental import pallas as pl
from jax.experimental.pallas import tpu as pltpu
from jax.experimental.pallas import tpu_sc as plsc

INF = 1e20
NCORE, NSUB, LANES = 2, 16, 16
NW = NCORE * NSUB
CH = 64
NPAD = 512


def _make_sc_kernel(B, C, NC, LT, LP, LC, LQ, D):
    NB = B // NW
    BCT = NB * C
    NCH = BCT // CH
    assert B % NW == 0 and BCT % CH == 0 and D % (2 * LANES) == 0
    KK = D // (2 * LANES)

    def body(table, qidx, tidx, pidx, cidx, wt, wp, wc, amask, out,
             sidx_q, sidx_t, sidx_p, sidx_c, acc_q, acc_t, acc_p, acc_c,
             qf32, dot_t, dot_p, dot_tmp, score, wbuf, sem, isem):
        wid = lax.axis_index("s") * NCORE + lax.axis_index("c")
        base_bc = wid * BCT

        cp_q = pltpu.async_copy(
            qidx.at[pl.ds(wid * (LQ * NB), LQ * NB)], sidx_q, isem)
        cp_t = pltpu.async_copy(
            tidx.at[pl.ds(wid * (LT * BCT), LT * BCT)], sidx_t, isem)
        cp_p = pltpu.async_copy(
            pidx.at[pl.ds(wid * (LP * BCT), LP * BCT)], sidx_p, isem)
        cp_c = pltpu.async_copy(
            cidx.at[pl.ds(wid * (NC * LC * BCT), NC * LC * BCT)], sidx_c, isem)

        zero32 = jnp.zeros((2 * LANES,), jnp.bfloat16)

        def zq(i, _):
            acc_q[i // KK, pl.ds((i % KK) * 2 * LANES, 2 * LANES)] = zero32
            return 0
        lax.fori_loop(0, NB * KK, zq, 0)

        def zbags(ref, nrows):
            def zstep(i, _):
                ref[i // KK, pl.ds((i % KK) * 2 * LANES, 2 * LANES)] = zero32
                return 0
            lax.fori_loop(0, nrows * KK, zstep, 0)

        zbags(acc_t, BCT)
        zbags(acc_p, BCT)
        zbags(acc_c, NC * BCT)

        cp_q.wait()
        cp_t.wait()
        cp_p.wait()
        cp_c.wait()

        def qfire(s, _):
            pltpu.async_copy(
                table.at[sidx_q.at[pl.ds(s * NB, NB)]], acc_q, sem, add=True)
            return 0
        lax.fori_loop(0, LQ, qfire, 0)

        def bag_fire(sidx, acc, nstreams):
            def fire(r, _):
                pltpu.async_copy(
                    table.at[sidx.at[pl.ds(r * CH, CH)]],
                    acc.at[pl.ds((r % NCH) * CH, CH), :], sem, add=True)
                return 0
            lax.fori_loop(0, nstreams, fire, 0)

        bag_fire(sidx_t, acc_t, LT * NCH)
        bag_fire(sidx_p, acc_p, LP * NCH)

        def cfire(r, _):
            n = r // (LC * NCH)
            cidx_ = r % NCH
            pltpu.async_copy(
                table.at[sidx_c.at[pl.ds(r * CH, CH)]],
                acc_c.at[pl.ds(n * BCT + cidx_ * CH, CH), :], sem, add=True)
            return 0
        lax.fori_loop(0, NC * LC * NCH, cfire, 0)

        def drain_q(i, _):
            pltpu.make_async_copy(table.at[pl.ds(0, NB)], acc_q, sem).wait()
            return 0
        lax.fori_loop(0, LQ, drain_q, 0)

        def drain_bag(i, _):
            pltpu.make_async_copy(
                table.at[pl.ds(0, CH)],
                acc_t.at[pl.ds(0, CH), :], sem).wait()
            return 0
        lax.fori_loop(0, (LT + LP + NC * LC) * NCH, drain_bag, 0)

        def uq(i, _):
            b = i // KK
            k = i % KK
            q2 = acc_q[b, pl.ds(k * 2 * LANES, 2 * LANES)]
            ql, qh = plsc.unpack(q2, format=plsc.PackFormat.INTERLEAVED)
            qf32[b, pl.ds(k * 2 * LANES, LANES)] = ql
            qf32[b, pl.ds(k * 2 * LANES + LANES, LANES)] = qh
            return 0
        lax.fori_loop(0, NB * KK, uq, 0)

        lane = lax.iota(jnp.int32, LANES)

        def dots(acc, blk, dotbuf):
            def dstep(g, _):
                s_vec = jnp.zeros((LANES,), jnp.float32)
                for j in range(LANES):
                    bag = g * LANES + j
                    b_loc = bag // C
                    prod = None
                    for k in range(KK):
                        a2 = acc[blk * BCT + bag,
                                 pl.ds(k * 2 * LANES, 2 * LANES)]
                        al, ah = plsc.unpack(
                            a2, format=plsc.PackFormat.INTERLEAVED)
                        ql = qf32[b_loc, pl.ds(k * 2 * LANES, LANES)]
                        qh = qf32[b_loc, pl.ds(k * 2 * LANES + LANES, LANES)]
                        p_ = al * ql + ah * qh
                        prod = p_ if prod is None else prod + p_
                    r = jnp.sum(prod)
                    s_vec = jnp.where(lane == j, r, s_vec)
                dotbuf[pl.ds(g * LANES, LANES)] = s_vec
                return 0
            lax.fori_loop(0, BCT // LANES, dstep, 0)

        dots(acc_t, 0, dot_t)
        dots(acc_p, 0, dot_p)

        def zscore(k, _):
            score[pl.ds(k * LANES, LANES)] = jnp.zeros((LANES,), jnp.float32)
            return 0
        lax.fori_loop(0, BCT // LANES, zscore, 0)

        for n in range(NC):
            dots(acc_c, n, dot_tmp)
            pltpu.sync_copy(wc.at[pl.ds(n * (B * C) + base_bc, BCT)], wbuf)

            def comb(k, _):
                sl = pl.ds(k * LANES, LANES)
                score[sl] = score[sl] + wbuf[sl] * dot_tmp[sl]
                return 0
            lax.fori_loop(0, BCT // LANES, comb, 0)

        pltpu.sync_copy(wt.at[pl.ds(base_bc, BCT)], wbuf)

        def combt(k, _):
            sl = pl.ds(k * LANES, LANES)
            score[sl] = score[sl] + wbuf[sl] * dot_t[sl]
            return 0
        lax.fori_loop(0, BCT // LANES, combt, 0)

        pltpu.sync_copy(wp.at[pl.ds(base_bc, BCT)], wbuf)

        def combp(k, _):
            sl = pl.ds(k * LANES, LANES)
            score[sl] = score[sl] + wbuf[sl] * dot_p[sl]
            return 0
        lax.fori_loop(0, BCT // LANES, combp, 0)

        pltpu.sync_copy(amask.at[pl.ds(base_bc, BCT)], wbuf)

        def maskstep(k, _):
            sl = pl.ds(k * LANES, LANES)
            m = wbuf[sl]
            score[sl] = m * score[sl] - (1.0 - m) * INF
            return 0
        lax.fori_loop(0, BCT // LANES, maskstep, 0)

        pltpu.sync_copy(score, out.at[pl.ds(base_bc, BCT)])

    mesh = plsc.VectorSubcoreMesh(
        core_axis_name="c", subcore_axis_name="s",
        num_cores=NCORE, num_subcores=NSUB)
    return pl.kernel(
        body,
        out_type=jax.ShapeDtypeStruct((B * C,), jnp.float32),
        mesh=mesh,
        compiler_params=pltpu.CompilerParams(
            use_tc_tiling_on_sc=False, needs_layout_passes=False),
        scratch_types=[
            pltpu.VMEM((LQ * NB,), jnp.int32),
            pltpu.VMEM((LT * BCT,), jnp.int32),
            pltpu.VMEM((LP * BCT,), jnp.int32),
            pltpu.VMEM((NC * LC * BCT,), jnp.int32),
            pltpu.VMEM((NB, D), jnp.bfloat16),
            pltpu.VMEM((BCT, D), jnp.bfloat16),
            pltpu.VMEM((BCT, D), jnp.bfloat16),
            pltpu.VMEM((NC * BCT, D), jnp.bfloat16),
            pltpu.VMEM((NB, D), jnp.float32),
            pltpu.VMEM((BCT,), jnp.float32),
            pltpu.VMEM((BCT,), jnp.float32),
            pltpu.VMEM((BCT,), jnp.float32),
            pltpu.VMEM((BCT,), jnp.float32),
            pltpu.VMEM((BCT,), jnp.float32),
            pltpu.SemaphoreType.DMA,
            pltpu.SemaphoreType.DMA,
        ],
    )


def kernel(emb_table, queries, query_lengths, ans_num,
           x_type_bow, x_type_bow_len, x_path_bow, x_path_bow_len,
           x_ctx_ent, x_ctx_ent_len, x_ctx_ent_num):
    V, D = emb_table.shape
    B, LQ = queries.shape
    C, LT = x_type_bow.shape[1], x_type_bow.shape[2]
    LP = x_path_bow.shape[2]
    NC, LC = x_ctx_ent.shape[2], x_ctx_ent.shape[3]
    BC = B * C
    NB = B // NW
    BCT = NB * C
    NCH = BCT // CH
    f32 = jnp.float32

    def pad_rows(shape):
        n = 1
        for d in shape:
            n *= d
        return (V + jnp.arange(n, dtype=jnp.int32) % NPAD).reshape(shape)

    qm = jnp.where(jnp.arange(LQ)[None, :] < query_lengths[:, None],
                   queries, pad_rows((B, LQ)))
    qidx = qm.reshape(NW, NB, LQ).transpose(0, 2, 1).reshape(-1)
    tlen = x_type_bow_len.reshape(-1)
    tm = jnp.where(jnp.arange(LT)[None, :] < tlen[:, None],
                   x_type_bow.reshape(-1, LT), pad_rows((BC, LT)))
    tidx = tm.reshape(NW, NCH, CH, LT).transpose(0, 3, 1, 2).reshape(-1)
    plen = x_path_bow_len.reshape(-1)
    pm = jnp.where(jnp.arange(LP)[None, :] < plen[:, None],
                   x_path_bow.reshape(-1, LP), pad_rows((BC, LP)))
    pidx = pm.reshape(NW, NCH, CH, LP).transpose(0, 3, 1, 2).reshape(-1)
    clen = x_ctx_ent_len.reshape(-1)
    cm = jnp.where(jnp.arange(LC)[None, :] < clen[:, None],
                   x_ctx_ent.reshape(-1, LC), pad_rows((BC * NC, LC)))
    cidx = (cm.reshape(NW, NCH, CH, NC, LC)
            .transpose(0, 3, 4, 1, 2).reshape(-1))

    qinv = 1.0 / query_lengths.astype(f32)
    qinv_bc = jnp.broadcast_to(qinv[:, None], (B, C)).reshape(-1)
    wt = qinv_bc / tlen.astype(f32)
    wp = qinv_bc / plen.astype(f32)
    cnum = x_ctx_ent_num.reshape(-1)
    nmask = (jnp.arange(NC)[None, :] < cnum[:, None]).astype(f32)
    wc = (nmask * qinv_bc[:, None]
          / (x_ctx_ent_len.reshape(BC, NC).astype(f32)
             * cnum[:, None].astype(f32))).T.reshape(-1)
    amask = (jnp.arange(C)[None, :] < ans_num[:, None]).astype(f32).reshape(-1)

    table_bf = emb_table.astype(jnp.bfloat16)
    table_p = jnp.concatenate(
        [table_bf, jnp.zeros((NPAD, D), jnp.bfloat16)], axis=0)
    fn = _make_sc_kernel(B, C, NC, LT, LP, LC, LQ, D)
    score = fn(table_p, qidx, tidx, pidx, cidx, wt, wp, wc, amask)
    return score.reshape(B, C)


# speedup vs baseline: 1.0627x; 1.0627x over previous
import jax
import jax.numpy as jnp
from jax import lax
from jax.experimental import pallas as pl
from jax.experimental.pallas import tpu as pltpu
from jax.experimental.pallas import tpu_sc as plsc

INF = 1e20
NCORE, NSUB, LANES = 2, 16, 16
NW = NCORE * NSUB
CH = 80
NPAD = 512


def _make_sc_kernel(B, C, NC, LT, LP, LC, LQ, V, D, D2):
    NB = B // NW
    BCT = NB * C
    NCH = BCT // CH
    assert B % NW == 0 and BCT % CH == 0 and D % LANES == 0

    def body(table, qidx, tidx, pidx, cidx, wt, wp, wc, amask, out,
             sidx_q, sidx_t, sidx_p, sidx_c, acc_q, acc_bc,
             dot_t, dot_p, dot_tmp, score, wbuf, sem):
        wid = lax.axis_index("s") * NCORE + lax.axis_index("c")

        def drain(n, dst_rows):
            def dstep(i, _):
                pltpu.make_async_copy(
                    table.at[pl.ds(0, dst_rows)],
                    acc_bc.at[pl.ds(0, dst_rows), :] if dst_rows == CH
                    else acc_q, sem).wait()
                return 0
            lax.fori_loop(0, n, dstep, 0)

        pltpu.sync_copy(qidx.at[pl.ds(wid * (LQ * NB), LQ * NB)], sidx_q)
        pltpu.async_copy(table.at[sidx_q.at[pl.ds(0, NB)]], acc_q, sem).wait()

        def qstep(s, _):
            pltpu.async_copy(
                table.at[sidx_q.at[pl.ds(s * NB, NB)]], acc_q, sem, add=True)
            return 0
        lax.fori_loop(1, LQ, qstep, 0)
        drain(LQ - 1, NB)

        def bag_phase(idx_hbm, blk, nrows, sidx, dotbuf):
            pltpu.sync_copy(
                idx_hbm.at[pl.ds(blk * (nrows * BCT), nrows * BCT)], sidx)

            def c0(cidx_, _):
                pltpu.async_copy(
                    table.at[sidx.at[pl.ds(cidx_ * CH, CH)]],
                    acc_bc.at[pl.ds(cidx_ * CH, CH), :], sem)
                return 0
            lax.fori_loop(0, NCH, c0, 0)
            drain(NCH, CH)

            def cadd(i, _):
                r = NCH + i
                cidx_ = i % NCH
                pltpu.async_copy(
                    table.at[sidx.at[pl.ds(r * CH, CH)]],
                    acc_bc.at[pl.ds(cidx_ * CH, CH), :], sem, add=True)
                return 0
            lax.fori_loop(0, (nrows - 1) * NCH, cadd, 0)
            drain((nrows - 1) * NCH, CH)

            lane = lax.iota(jnp.int32, LANES)

            def dstep(g, _):
                s_vec = jnp.zeros((LANES,), jnp.float32)
                for j in range(LANES):
                    bag = g * LANES + j
                    b_loc = bag // C
                    prod = None
                    for k in range(D // (2 * LANES)):
                        a2 = acc_bc[bag, pl.ds(k * 2 * LANES, 2 * LANES)]
                        q2 = acc_q[b_loc, pl.ds(k * 2 * LANES, 2 * LANES)]
                        al, ah = plsc.unpack(a2, format=plsc.PackFormat.INTERLEAVED)
                        ql, qh = plsc.unpack(q2, format=plsc.PackFormat.INTERLEAVED)
                        p_ = al * ql + ah * qh
                        prod = p_ if prod is None else prod + p_
                    r = jnp.sum(prod)
                    s_vec = jnp.where(lane == j, r, s_vec)
                dotbuf[pl.ds(g * LANES, LANES)] = s_vec
                return 0
            lax.fori_loop(0, BCT // LANES, dstep, 0)

        bag_phase(tidx, wid, LT, sidx_t, dot_t)
        bag_phase(pidx, wid, LP, sidx_p, dot_p)

        def zstep(k, _):
            score[pl.ds(k * LANES, LANES)] = jnp.zeros((LANES,), jnp.float32)
            return 0
        lax.fori_loop(0, BCT // LANES, zstep, 0)

        for n in range(NC):
            bag_phase(cidx, wid * NC + n, LC, sidx_c, dot_tmp)
            pltpu.sync_copy(wc.at[pl.ds(n * (B * C) + wid * BCT, BCT)], wbuf)

            def comb(k, _):
                sl = pl.ds(k * LANES, LANES)
                score[sl] = score[sl] + wbuf[sl] * dot_tmp[sl]
                return 0
            lax.fori_loop(0, BCT // LANES, comb, 0)

        base_bc = wid * BCT
        pltpu.sync_copy(wt.at[pl.ds(base_bc, BCT)], wbuf)

        def combt(k, _):
            sl = pl.ds(k * LANES, LANES)
            score[sl] = score[sl] + wbuf[sl] * dot_t[sl]
            return 0
        lax.fori_loop(0, BCT // LANES, combt, 0)

        pltpu.sync_copy(wp.at[pl.ds(base_bc, BCT)], wbuf)

        def combp(k, _):
            sl = pl.ds(k * LANES, LANES)
            score[sl] = score[sl] + wbuf[sl] * dot_p[sl]
            return 0
        lax.fori_loop(0, BCT // LANES, combp, 0)

        pltpu.sync_copy(amask.at[pl.ds(base_bc, BCT)], wbuf)

        def maskstep(k, _):
            sl = pl.ds(k * LANES, LANES)
            m = wbuf[sl]
            score[sl] = m * score[sl] - (1.0 - m) * INF
            return 0
        lax.fori_loop(0, BCT // LANES, maskstep, 0)

        pltpu.sync_copy(score, out.at[pl.ds(base_bc, BCT)])

    mesh = plsc.VectorSubcoreMesh(
        core_axis_name="c", subcore_axis_name="s",
        num_cores=NCORE, num_subcores=NSUB)
    return pl.kernel(
        body,
        out_type=jax.ShapeDtypeStruct((B * C,), jnp.float32),
        mesh=mesh,
        compiler_params=pltpu.CompilerParams(
            use_tc_tiling_on_sc=False, needs_layout_passes=False),
        scratch_types=[
            pltpu.VMEM((LQ * NB,), jnp.int32),
            pltpu.VMEM((LT * BCT,), jnp.int32),
            pltpu.VMEM((LP * BCT,), jnp.int32),
            pltpu.VMEM((LC * BCT,), jnp.int32),
            pltpu.VMEM((NB, D2), jnp.bfloat16),
            pltpu.VMEM((BCT, D2), jnp.bfloat16),
            pltpu.VMEM((BCT,), jnp.float32),
            pltpu.VMEM((BCT,), jnp.float32),
            pltpu.VMEM((BCT,), jnp.float32),
            pltpu.VMEM((BCT,), jnp.float32),
            pltpu.VMEM((BCT,), jnp.float32),
            pltpu.SemaphoreType.DMA,
        ],
    )


def kernel(emb_table, queries, query_lengths, ans_num,
           x_type_bow, x_type_bow_len, x_path_bow, x_path_bow_len,
           x_ctx_ent, x_ctx_ent_len, x_ctx_ent_num):
    V, D = emb_table.shape
    B, LQ = queries.shape
    C, LT = x_type_bow.shape[1], x_type_bow.shape[2]
    LP = x_path_bow.shape[2]
    NC, LC = x_ctx_ent.shape[2], x_ctx_ent.shape[3]
    BC = B * C
    NB = B // NW
    BCT = NB * C
    NCH = BCT // CH
    f32 = jnp.float32

    def pad_rows(shape):
        n = 1
        for d in shape:
            n *= d
        return (V + jnp.arange(n, dtype=jnp.int32) % NPAD).reshape(shape)

    qm = jnp.where(jnp.arange(LQ)[None, :] < query_lengths[:, None],
                   queries, pad_rows((B, LQ)))
    qidx = qm.reshape(NW, NB, LQ).transpose(0, 2, 1).reshape(-1)
    tlen = x_type_bow_len.reshape(-1)
    tm = jnp.where(jnp.arange(LT)[None, :] < tlen[:, None],
                   x_type_bow.reshape(-1, LT), pad_rows((BC, LT)))
    tidx = tm.reshape(NW, NCH, CH, LT).transpose(0, 3, 1, 2).reshape(-1)
    plen = x_path_bow_len.reshape(-1)
    pm = jnp.where(jnp.arange(LP)[None, :] < plen[:, None],
                   x_path_bow.reshape(-1, LP), pad_rows((BC, LP)))
    pidx = pm.reshape(NW, NCH, CH, LP).transpose(0, 3, 1, 2).reshape(-1)
    clen = x_ctx_ent_len.reshape(-1)
    cm = jnp.where(jnp.arange(LC)[None, :] < clen[:, None],
                   x_ctx_ent.reshape(-1, LC), pad_rows((BC * NC, LC)))
    cidx = (cm.reshape(NW, NCH, CH, NC, LC)
            .transpose(0, 3, 4, 1, 2).reshape(-1))

    qinv = 1.0 / query_lengths.astype(f32)
    qinv_bc = jnp.broadcast_to(qinv[:, None], (B, C)).reshape(-1)
    wt = qinv_bc / tlen.astype(f32)
    wp = qinv_bc / plen.astype(f32)
    cnum = x_ctx_ent_num.reshape(-1)
    nmask = (jnp.arange(NC)[None, :] < cnum[:, None]).astype(f32)
    wc = (nmask * qinv_bc[:, None]
          / (x_ctx_ent_len.reshape(BC, NC).astype(f32)
             * cnum[:, None].astype(f32))).T.reshape(-1)
    amask = (jnp.arange(C)[None, :] < ans_num[:, None]).astype(f32).reshape(-1)

    D2 = D
    table_bf = emb_table.astype(jnp.bfloat16)
    table_p = jnp.concatenate(
        [table_bf, jnp.zeros((NPAD, D2), jnp.bfloat16)], axis=0)
    fn = _make_sc_kernel(B, C, NC, LT, LP, LC, LQ, V + NPAD, D, D2)
    score = fn(table_p, qidx, tidx, pidx, cidx, wt, wp, wc, amask)
    return score.reshape(B, C)


# speedup vs baseline: 1.0637x; 1.0010x over previous
import jax
import jax.numpy as jnp
from jax import lax
from jax.experimental import pallas as pl
from jax.experimental.pallas import tpu as pltpu
from jax.experimental.pallas import tpu_sc as plsc

INF = 1e20
NCORE, NSUB, LANES = 2, 16, 16
NW = NCORE * NSUB
CH = 80
NPAD = 512


def _make_sc_kernel(B, C, NC, LT, LP, LC, LQ, V, D, D2):
    NB = B // NW
    BCT = NB * C
    NCH = BCT // CH
    assert B % NW == 0 and BCT % CH == 0 and D % LANES == 0

    def body(table, qidx, tidx, pidx, cidx, wt, wp, wc, amask, out,
             sidx_q, sidx_t, sidx_p, sidx_c, acc_q, acc_bc,
             dot_t, dot_p, dot_tmp, score, wbuf, sem):
        wid = lax.axis_index("s") * NCORE + lax.axis_index("c")

        def drain(n, dst_rows):
            def dstep(i, _):
                pltpu.make_async_copy(
                    table.at[pl.ds(0, dst_rows)],
                    acc_bc.at[pl.ds(0, dst_rows), :] if dst_rows == CH
                    else acc_q, sem).wait()
                return 0
            lax.fori_loop(0, n, dstep, 0)

        pltpu.sync_copy(qidx.at[pl.ds(wid * (LQ * NB), LQ * NB)], sidx_q)
        pltpu.async_copy(table.at[sidx_q.at[pl.ds(0, NB)]], acc_q, sem).wait()

        def qstep(s, _):
            pltpu.async_copy(
                table.at[sidx_q.at[pl.ds(s * NB, NB)]], acc_q, sem)
            return 0
        lax.fori_loop(1, LQ, qstep, 0)
        drain(LQ - 1, NB)

        def bag_phase(idx_hbm, blk, nrows, sidx, dotbuf):
            pltpu.sync_copy(
                idx_hbm.at[pl.ds(blk * (nrows * BCT), nrows * BCT)], sidx)

            def c0(cidx_, _):
                pltpu.async_copy(
                    table.at[sidx.at[pl.ds(cidx_ * CH, CH)]],
                    acc_bc.at[pl.ds(cidx_ * CH, CH), :], sem)
                return 0
            lax.fori_loop(0, NCH, c0, 0)
            drain(NCH, CH)

            def cadd(i, _):
                r = NCH + i
                cidx_ = i % NCH
                pltpu.async_copy(
                    table.at[sidx.at[pl.ds(r * CH, CH)]],
                    acc_bc.at[pl.ds(cidx_ * CH, CH), :], sem)
                return 0
            lax.fori_loop(0, (nrows - 1) * NCH, cadd, 0)
            drain((nrows - 1) * NCH, CH)

            lane = lax.iota(jnp.int32, LANES)

            def dstep(g, _):
                s_vec = jnp.zeros((LANES,), jnp.float32)
                for j in range(LANES):
                    bag = g * LANES + j
                    b_loc = bag // C
                    prod = None
                    for k in range(D // (2 * LANES)):
                        a2 = acc_bc[bag, pl.ds(k * 2 * LANES, 2 * LANES)]
                        q2 = acc_q[b_loc, pl.ds(k * 2 * LANES, 2 * LANES)]
                        al, ah = plsc.unpack(a2, format=plsc.PackFormat.INTERLEAVED)
                        ql, qh = plsc.unpack(q2, format=plsc.PackFormat.INTERLEAVED)
                        p_ = al * ql + ah * qh
                        prod = p_ if prod is None else prod + p_
                    r = jnp.sum(prod)
                    s_vec = jnp.where(lane == j, r, s_vec)
                dotbuf[pl.ds(g * LANES, LANES)] = s_vec
                return 0
            lax.fori_loop(0, BCT // LANES, dstep, 0)

        bag_phase(tidx, wid, LT, sidx_t, dot_t)
        bag_phase(pidx, wid, LP, sidx_p, dot_p)

        def zstep(k, _):
            score[pl.ds(k * LANES, LANES)] = jnp.zeros((LANES,), jnp.float32)
            return 0
        lax.fori_loop(0, BCT // LANES, zstep, 0)

        for n in range(NC):
            bag_phase(cidx, wid * NC + n, LC, sidx_c, dot_tmp)
            pltpu.sync_copy(wc.at[pl.ds(n * (B * C) + wid * BCT, BCT)], wbuf)

            def comb(k, _):
                sl = pl.ds(k * LANES, LANES)
                score[sl] = score[sl] + wbuf[sl] * dot_tmp[sl]
                return 0
            lax.fori_loop(0, BCT // LANES, comb, 0)

        base_bc = wid * BCT
        pltpu.sync_copy(wt.at[pl.ds(base_bc, BCT)], wbuf)

        def combt(k, _):
            sl = pl.ds(k * LANES, LANES)
            score[sl] = score[sl] + wbuf[sl] * dot_t[sl]
            return 0
        lax.fori_loop(0, BCT // LANES, combt, 0)

        pltpu.sync_copy(wp.at[pl.ds(base_bc, BCT)], wbuf)

        def combp(k, _):
            sl = pl.ds(k * LANES, LANES)
            score[sl] = score[sl] + wbuf[sl] * dot_p[sl]
            return 0
        lax.fori_loop(0, BCT // LANES, combp, 0)

        pltpu.sync_copy(amask.at[pl.ds(base_bc, BCT)], wbuf)

        def maskstep(k, _):
            sl = pl.ds(k * LANES, LANES)
            m = wbuf[sl]
            score[sl] = m * score[sl] - (1.0 - m) * INF
            return 0
        lax.fori_loop(0, BCT // LANES, maskstep, 0)

        pltpu.sync_copy(score, out.at[pl.ds(base_bc, BCT)])

    mesh = plsc.VectorSubcoreMesh(
        core_axis_name="c", subcore_axis_name="s",
        num_cores=NCORE, num_subcores=NSUB)
    return pl.kernel(
        body,
        out_type=jax.ShapeDtypeStruct((B * C,), jnp.float32),
        mesh=mesh,
        compiler_params=pltpu.CompilerParams(
            use_tc_tiling_on_sc=False, needs_layout_passes=False),
        scratch_types=[
            pltpu.VMEM((LQ * NB,), jnp.int32),
            pltpu.VMEM((LT * BCT,), jnp.int32),
            pltpu.VMEM((LP * BCT,), jnp.int32),
            pltpu.VMEM((LC * BCT,), jnp.int32),
            pltpu.VMEM((NB, D2), jnp.bfloat16),
            pltpu.VMEM((BCT, D2), jnp.bfloat16),
            pltpu.VMEM((BCT,), jnp.float32),
            pltpu.VMEM((BCT,), jnp.float32),
            pltpu.VMEM((BCT,), jnp.float32),
            pltpu.VMEM((BCT,), jnp.float32),
            pltpu.VMEM((BCT,), jnp.float32),
            pltpu.SemaphoreType.DMA,
        ],
    )


def kernel(emb_table, queries, query_lengths, ans_num,
           x_type_bow, x_type_bow_len, x_path_bow, x_path_bow_len,
           x_ctx_ent, x_ctx_ent_len, x_ctx_ent_num):
    V, D = emb_table.shape
    B, LQ = queries.shape
    C, LT = x_type_bow.shape[1], x_type_bow.shape[2]
    LP = x_path_bow.shape[2]
    NC, LC = x_ctx_ent.shape[2], x_ctx_ent.shape[3]
    BC = B * C
    NB = B // NW
    BCT = NB * C
    NCH = BCT // CH
    f32 = jnp.float32

    def pad_rows(shape):
        n = 1
        for d in shape:
            n *= d
        return (V + jnp.arange(n, dtype=jnp.int32) % NPAD).reshape(shape)

    qm = jnp.where(jnp.arange(LQ)[None, :] < query_lengths[:, None],
                   queries, pad_rows((B, LQ)))
    qidx = qm.reshape(NW, NB, LQ).transpose(0, 2, 1).reshape(-1)
    tlen = x_type_bow_len.reshape(-1)
    tm = jnp.where(jnp.arange(LT)[None, :] < tlen[:, None],
                   x_type_bow.reshape(-1, LT), pad_rows((BC, LT)))
    tidx = tm.reshape(NW, NCH, CH, LT).transpose(0, 3, 1, 2).reshape(-1)
    plen = x_path_bow_len.reshape(-1)
    pm = jnp.where(jnp.arange(LP)[None, :] < plen[:, None],
                   x_path_bow.reshape(-1, LP), pad_rows((BC, LP)))
    pidx = pm.reshape(NW, NCH, CH, LP).transpose(0, 3, 1, 2).reshape(-1)
    clen = x_ctx_ent_len.reshape(-1)
    cm = jnp.where(jnp.arange(LC)[None, :] < clen[:, None],
                   x_ctx_ent.reshape(-1, LC), pad_rows((BC * NC, LC)))
    cidx = (cm.reshape(NW, NCH, CH, NC, LC)
            .transpose(0, 3, 4, 1, 2).reshape(-1))

    qinv = 1.0 / query_lengths.astype(f32)
    qinv_bc = jnp.broadcast_to(qinv[:, None], (B, C)).reshape(-1)
    wt = qinv_bc / tlen.astype(f32)
    wp = qinv_bc / plen.astype(f32)
    cnum = x_ctx_ent_num.reshape(-1)
    nmask = (jnp.arange(NC)[None, :] < cnum[:, None]).astype(f32)
    wc = (nmask * qinv_bc[:, None]
          / (x_ctx_ent_len.reshape(BC, NC).astype(f32)
             * cnum[:, None].astype(f32))).T.reshape(-1)
    amask = (jnp.arange(C)[None, :] < ans_num[:, None]).astype(f32).reshape(-1)

    D2 = D
    table_bf = emb_table.astype(jnp.bfloat16)
    table_p = jnp.concatenate(
        [table_bf, jnp.zeros((NPAD, D2), jnp.bfloat16)], axis=0)
    fn = _make_sc_kernel(B, C, NC, LT, LP, LC, LQ, V + NPAD, D, D2)
    score = fn(table_p, qidx, tidx, pidx, cidx, wt, wp, wc, amask)
    return score.reshape(B, C)


# speedup vs baseline: 1.2652x; 1.1893x over previous
import jax
import jax.numpy as jnp
from jax import lax
from jax.experimental import pallas as pl
from jax.experimental.pallas import tpu as pltpu
from jax.experimental.pallas import tpu_sc as plsc

INF = 1e20
NCORE, NSUB, LANES = 2, 16, 16
NW = NCORE * NSUB
CH = 80
NPAD = 8192


def _make_sc_kernel(B, C, NC, LT, LP, LC, LQ, V, D, D2):
    NB = B // NW
    BCT = NB * C
    NCH = BCT // CH
    assert B % NW == 0 and BCT % CH == 0 and D % LANES == 0

    def body(table, qidx, tidx, pidx, cidx, wt, wp, wc, amask, out,
             sidx_q, sidx_t, sidx_p, sidx_c, acc_q, acc_bc,
             dot_t, dot_p, dot_tmp, score, wbuf, sem):
        wid = lax.axis_index("s") * NCORE + lax.axis_index("c")

        def drain(n, dst_rows):
            def dstep(i, _):
                pltpu.make_async_copy(
                    table.at[pl.ds(0, dst_rows)],
                    acc_bc.at[pl.ds(0, dst_rows), :] if dst_rows == CH
                    else acc_q, sem).wait()
                return 0
            lax.fori_loop(0, n, dstep, 0)

        pltpu.sync_copy(qidx.at[pl.ds(wid * (LQ * NB), LQ * NB)], sidx_q)
        pltpu.async_copy(table.at[sidx_q.at[pl.ds(0, NB)]], acc_q, sem).wait()

        def qstep(s, _):
            pltpu.async_copy(
                table.at[sidx_q.at[pl.ds(s * NB, NB)]], acc_q, sem, add=True)
            return 0
        lax.fori_loop(1, LQ, qstep, 0)
        drain(LQ - 1, NB)

        def bag_phase(idx_hbm, blk, nrows, sidx, dotbuf):
            pltpu.sync_copy(
                idx_hbm.at[pl.ds(blk * (nrows * BCT), nrows * BCT)], sidx)

            def c0(cidx_, _):
                pltpu.async_copy(
                    table.at[sidx.at[pl.ds(cidx_ * CH, CH)]],
                    acc_bc.at[pl.ds(cidx_ * CH, CH), :], sem)
                return 0
            lax.fori_loop(0, NCH, c0, 0)
            drain(NCH, CH)

            def cadd(i, _):
                r = NCH + i
                cidx_ = i % NCH
                pltpu.async_copy(
                    table.at[sidx.at[pl.ds(r * CH, CH)]],
                    acc_bc.at[pl.ds(cidx_ * CH, CH), :], sem, add=True)
                return 0
            lax.fori_loop(0, (nrows - 1) * NCH, cadd, 0)
            drain((nrows - 1) * NCH, CH)

            lane = lax.iota(jnp.int32, LANES)

            def dstep(g, _):
                s_vec = jnp.zeros((LANES,), jnp.float32)
                for j in range(LANES):
                    bag = g * LANES + j
                    b_loc = bag // C
                    prod = None
                    for k in range(D // (2 * LANES)):
                        a2 = acc_bc[bag, pl.ds(k * 2 * LANES, 2 * LANES)]
                        q2 = acc_q[b_loc, pl.ds(k * 2 * LANES, 2 * LANES)]
                        al, ah = plsc.unpack(a2, format=plsc.PackFormat.INTERLEAVED)
                        ql, qh = plsc.unpack(q2, format=plsc.PackFormat.INTERLEAVED)
                        p_ = al * ql + ah * qh
                        prod = p_ if prod is None else prod + p_
                    r = jnp.sum(prod)
                    s_vec = jnp.where(lane == j, r, s_vec)
                dotbuf[pl.ds(g * LANES, LANES)] = s_vec
                return 0
            lax.fori_loop(0, BCT // LANES, dstep, 0)

        bag_phase(tidx, wid, LT, sidx_t, dot_t)
        bag_phase(pidx, wid, LP, sidx_p, dot_p)

        def zstep(k, _):
            score[pl.ds(k * LANES, LANES)] = jnp.zeros((LANES,), jnp.float32)
            return 0
        lax.fori_loop(0, BCT // LANES, zstep, 0)

        for n in range(NC):
            bag_phase(cidx, wid * NC + n, LC, sidx_c, dot_tmp)
            pltpu.sync_copy(wc.at[pl.ds(n * (B * C) + wid * BCT, BCT)], wbuf)

            def comb(k, _):
                sl = pl.ds(k * LANES, LANES)
                score[sl] = score[sl] + wbuf[sl] * dot_tmp[sl]
                return 0
            lax.fori_loop(0, BCT // LANES, comb, 0)

        base_bc = wid * BCT
        pltpu.sync_copy(wt.at[pl.ds(base_bc, BCT)], wbuf)

        def combt(k, _):
            sl = pl.ds(k * LANES, LANES)
            score[sl] = score[sl] + wbuf[sl] * dot_t[sl]
            return 0
        lax.fori_loop(0, BCT // LANES, combt, 0)

        pltpu.sync_copy(wp.at[pl.ds(base_bc, BCT)], wbuf)

        def combp(k, _):
            sl = pl.ds(k * LANES, LANES)
            score[sl] = score[sl] + wbuf[sl] * dot_p[sl]
            return 0
        lax.fori_loop(0, BCT // LANES, combp, 0)

        pltpu.sync_copy(amask.at[pl.ds(base_bc, BCT)], wbuf)

        def maskstep(k, _):
            sl = pl.ds(k * LANES, LANES)
            m = wbuf[sl]
            score[sl] = m * score[sl] - (1.0 - m) * INF
            return 0
        lax.fori_loop(0, BCT // LANES, maskstep, 0)

        pltpu.sync_copy(score, out.at[pl.ds(base_bc, BCT)])

    mesh = plsc.VectorSubcoreMesh(
        core_axis_name="c", subcore_axis_name="s",
        num_cores=NCORE, num_subcores=NSUB)
    return pl.kernel(
        body,
        out_type=jax.ShapeDtypeStruct((B * C,), jnp.float32),
        mesh=mesh,
        compiler_params=pltpu.CompilerParams(
            use_tc_tiling_on_sc=False, needs_layout_passes=False),
        scratch_types=[
            pltpu.VMEM((LQ * NB,), jnp.int32),
            pltpu.VMEM((LT * BCT,), jnp.int32),
            pltpu.VMEM((LP * BCT,), jnp.int32),
            pltpu.VMEM((LC * BCT,), jnp.int32),
            pltpu.VMEM((NB, D2), jnp.bfloat16),
            pltpu.VMEM((BCT, D2), jnp.bfloat16),
            pltpu.VMEM((BCT,), jnp.float32),
            pltpu.VMEM((BCT,), jnp.float32),
            pltpu.VMEM((BCT,), jnp.float32),
            pltpu.VMEM((BCT,), jnp.float32),
            pltpu.VMEM((BCT,), jnp.float32),
            pltpu.SemaphoreType.DMA,
        ],
    )


def kernel(emb_table, queries, query_lengths, ans_num,
           x_type_bow, x_type_bow_len, x_path_bow, x_path_bow_len,
           x_ctx_ent, x_ctx_ent_len, x_ctx_ent_num):
    V, D = emb_table.shape
    B, LQ = queries.shape
    C, LT = x_type_bow.shape[1], x_type_bow.shape[2]
    LP = x_path_bow.shape[2]
    NC, LC = x_ctx_ent.shape[2], x_ctx_ent.shape[3]
    BC = B * C
    NB = B // NW
    BCT = NB * C
    NCH = BCT // CH
    f32 = jnp.float32

    def pad_rows(shape):
        n = 1
        for d in shape:
            n *= d
        return (V + jnp.arange(n, dtype=jnp.int32) % NPAD).reshape(shape)

    qm = jnp.where(jnp.arange(LQ)[None, :] < query_lengths[:, None],
                   queries, pad_rows((B, LQ)))
    qidx = qm.reshape(NW, NB, LQ).transpose(0, 2, 1).reshape(-1)
    tlen = x_type_bow_len.reshape(-1)
    tm = jnp.where(jnp.arange(LT)[None, :] < tlen[:, None],
                   x_type_bow.reshape(-1, LT), pad_rows((BC, LT)))
    tidx = tm.reshape(NW, NCH, CH, LT).transpose(0, 3, 1, 2).reshape(-1)
    plen = x_path_bow_len.reshape(-1)
    pm = jnp.where(jnp.arange(LP)[None, :] < plen[:, None],
                   x_path_bow.reshape(-1, LP), pad_rows((BC, LP)))
    pidx = pm.reshape(NW, NCH, CH, LP).transpose(0, 3, 1, 2).reshape(-1)
    clen = x_ctx_ent_len.reshape(-1)
    cm = jnp.where(jnp.arange(LC)[None, :] < clen[:, None],
                   x_ctx_ent.reshape(-1, LC), pad_rows((BC * NC, LC)))
    cidx = (cm.reshape(NW, NCH, CH, NC, LC)
            .transpose(0, 3, 4, 1, 2).reshape(-1))

    qinv = 1.0 / query_lengths.astype(f32)
    qinv_bc = jnp.broadcast_to(qinv[:, None], (B, C)).reshape(-1)
    wt = qinv_bc / tlen.astype(f32)
    wp = qinv_bc / plen.astype(f32)
    cnum = x_ctx_ent_num.reshape(-1)
    nmask = (jnp.arange(NC)[None, :] < cnum[:, None]).astype(f32)
    wc = (nmask * qinv_bc[:, None]
          / (x_ctx_ent_len.reshape(BC, NC).astype(f32)
             * cnum[:, None].astype(f32))).T.reshape(-1)
    amask = (jnp.arange(C)[None, :] < ans_num[:, None]).astype(f32).reshape(-1)

    D2 = D
    table_bf = emb_table.astype(jnp.bfloat16)
    table_p = jnp.concatenate(
        [table_bf, jnp.zeros((NPAD, D2), jnp.bfloat16)], axis=0)
    fn = _make_sc_kernel(B, C, NC, LT, LP, LC, LQ, V + NPAD, D, D2)
    score = fn(table_p, qidx, tidx, pidx, cidx, wt, wp, wc, amask)
    return score.reshape(B, C)


# speedup vs baseline: 1.3058x; 1.0321x over previous
import jax
import jax.numpy as jnp
from jax import lax
from jax.experimental import pallas as pl
from jax.experimental.pallas import tpu as pltpu
from jax.experimental.pallas import tpu_sc as plsc

INF = 1e20
NCORE, NSUB, LANES = 2, 16, 16
NW = NCORE * NSUB
CH = 80
NPAD = 32768


def _make_sc_kernel(B, C, NC, LT, LP, LC, LQ, V, D, D2):
    NB = B // NW
    BCT = NB * C
    NCH = BCT // CH
    assert B % NW == 0 and BCT % CH == 0 and D % LANES == 0

    def body(table, qidx, tidx, pidx, cidx, wt, wp, wc, amask, out,
             sidx_q, sidx_t, sidx_p, sidx_c, acc_q, acc_bc,
             dot_t, dot_p, dot_tmp, score, wbuf, sem):
        wid = lax.axis_index("s") * NCORE + lax.axis_index("c")

        def drain(n, dst_rows):
            def dstep(i, _):
                pltpu.make_async_copy(
                    table.at[pl.ds(0, dst_rows)],
                    acc_bc.at[pl.ds(0, dst_rows), :] if dst_rows == CH
                    else acc_q, sem).wait()
                return 0
            lax.fori_loop(0, n, dstep, 0)

        pltpu.sync_copy(qidx.at[pl.ds(wid * (LQ * NB), LQ * NB)], sidx_q)
        pltpu.async_copy(table.at[sidx_q.at[pl.ds(0, NB)]], acc_q, sem).wait()

        def qstep(s, _):
            pltpu.async_copy(
                table.at[sidx_q.at[pl.ds(s * NB, NB)]], acc_q, sem, add=True)
            return 0
        lax.fori_loop(1, LQ, qstep, 0)
        drain(LQ - 1, NB)

        def bag_phase(idx_hbm, blk, nrows, sidx, dotbuf):
            pltpu.sync_copy(
                idx_hbm.at[pl.ds(blk * (nrows * BCT), nrows * BCT)], sidx)

            def c0(cidx_, _):
                pltpu.async_copy(
                    table.at[sidx.at[pl.ds(cidx_ * CH, CH)]],
                    acc_bc.at[pl.ds(cidx_ * CH, CH), :], sem)
                return 0
            lax.fori_loop(0, NCH, c0, 0)
            drain(NCH, CH)

            def cadd(i, _):
                r = NCH + i
                cidx_ = i % NCH
                pltpu.async_copy(
                    table.at[sidx.at[pl.ds(r * CH, CH)]],
                    acc_bc.at[pl.ds(cidx_ * CH, CH), :], sem, add=True)
                return 0
            lax.fori_loop(0, (nrows - 1) * NCH, cadd, 0)
            drain((nrows - 1) * NCH, CH)

            lane = lax.iota(jnp.int32, LANES)

            def dstep(g, _):
                s_vec = jnp.zeros((LANES,), jnp.float32)
                for j in range(LANES):
                    bag = g * LANES + j
                    b_loc = bag // C
                    prod = None
                    for k in range(D // (2 * LANES)):
                        a2 = acc_bc[bag, pl.ds(k * 2 * LANES, 2 * LANES)]
                        q2 = acc_q[b_loc, pl.ds(k * 2 * LANES, 2 * LANES)]
                        al, ah = plsc.unpack(a2, format=plsc.PackFormat.INTERLEAVED)
                        ql, qh = plsc.unpack(q2, format=plsc.PackFormat.INTERLEAVED)
                        p_ = al * ql + ah * qh
                        prod = p_ if prod is None else prod + p_
                    r = jnp.sum(prod)
                    s_vec = jnp.where(lane == j, r, s_vec)
                dotbuf[pl.ds(g * LANES, LANES)] = s_vec
                return 0
            lax.fori_loop(0, BCT // LANES, dstep, 0)

        bag_phase(tidx, wid, LT, sidx_t, dot_t)
        bag_phase(pidx, wid, LP, sidx_p, dot_p)

        def zstep(k, _):
            score[pl.ds(k * LANES, LANES)] = jnp.zeros((LANES,), jnp.float32)
            return 0
        lax.fori_loop(0, BCT // LANES, zstep, 0)

        for n in range(NC):
            bag_phase(cidx, wid * NC + n, LC, sidx_c, dot_tmp)
            pltpu.sync_copy(wc.at[pl.ds(n * (B * C) + wid * BCT, BCT)], wbuf)

            def comb(k, _):
                sl = pl.ds(k * LANES, LANES)
                score[sl] = score[sl] + wbuf[sl] * dot_tmp[sl]
                return 0
            lax.fori_loop(0, BCT // LANES, comb, 0)

        base_bc = wid * BCT
        pltpu.sync_copy(wt.at[pl.ds(base_bc, BCT)], wbuf)

        def combt(k, _):
            sl = pl.ds(k * LANES, LANES)
            score[sl] = score[sl] + wbuf[sl] * dot_t[sl]
            return 0
        lax.fori_loop(0, BCT // LANES, combt, 0)

        pltpu.sync_copy(wp.at[pl.ds(base_bc, BCT)], wbuf)

        def combp(k, _):
            sl = pl.ds(k * LANES, LANES)
            score[sl] = score[sl] + wbuf[sl] * dot_p[sl]
            return 0
        lax.fori_loop(0, BCT // LANES, combp, 0)

        pltpu.sync_copy(amask.at[pl.ds(base_bc, BCT)], wbuf)

        def maskstep(k, _):
            sl = pl.ds(k * LANES, LANES)
            m = wbuf[sl]
            score[sl] = m * score[sl] - (1.0 - m) * INF
            return 0
        lax.fori_loop(0, BCT // LANES, maskstep, 0)

        pltpu.sync_copy(score, out.at[pl.ds(base_bc, BCT)])

    mesh = plsc.VectorSubcoreMesh(
        core_axis_name="c", subcore_axis_name="s",
        num_cores=NCORE, num_subcores=NSUB)
    return pl.kernel(
        body,
        out_type=jax.ShapeDtypeStruct((B * C,), jnp.float32),
        mesh=mesh,
        compiler_params=pltpu.CompilerParams(
            use_tc_tiling_on_sc=False, needs_layout_passes=False),
        scratch_types=[
            pltpu.VMEM((LQ * NB,), jnp.int32),
            pltpu.VMEM((LT * BCT,), jnp.int32),
            pltpu.VMEM((LP * BCT,), jnp.int32),
            pltpu.VMEM((LC * BCT,), jnp.int32),
            pltpu.VMEM((NB, D2), jnp.bfloat16),
            pltpu.VMEM((BCT, D2), jnp.bfloat16),
            pltpu.VMEM((BCT,), jnp.float32),
            pltpu.VMEM((BCT,), jnp.float32),
            pltpu.VMEM((BCT,), jnp.float32),
            pltpu.VMEM((BCT,), jnp.float32),
            pltpu.VMEM((BCT,), jnp.float32),
            pltpu.SemaphoreType.DMA,
        ],
    )


def kernel(emb_table, queries, query_lengths, ans_num,
           x_type_bow, x_type_bow_len, x_path_bow, x_path_bow_len,
           x_ctx_ent, x_ctx_ent_len, x_ctx_ent_num):
    V, D = emb_table.shape
    B, LQ = queries.shape
    C, LT = x_type_bow.shape[1], x_type_bow.shape[2]
    LP = x_path_bow.shape[2]
    NC, LC = x_ctx_ent.shape[2], x_ctx_ent.shape[3]
    BC = B * C
    NB = B // NW
    BCT = NB * C
    NCH = BCT // CH
    f32 = jnp.float32

    def pad_rows(shape):
        n = 1
        for d in shape:
            n *= d
        return (V + jnp.arange(n, dtype=jnp.int32) % NPAD).reshape(shape)

    qm = jnp.where(jnp.arange(LQ)[None, :] < query_lengths[:, None],
                   queries, pad_rows((B, LQ)))
    qidx = qm.reshape(NW, NB, LQ).transpose(0, 2, 1).reshape(-1)
    tlen = x_type_bow_len.reshape(-1)
    tm = jnp.where(jnp.arange(LT)[None, :] < tlen[:, None],
                   x_type_bow.reshape(-1, LT), pad_rows((BC, LT)))
    tidx = tm.reshape(NW, NCH, CH, LT).transpose(0, 3, 1, 2).reshape(-1)
    plen = x_path_bow_len.reshape(-1)
    pm = jnp.where(jnp.arange(LP)[None, :] < plen[:, None],
                   x_path_bow.reshape(-1, LP), pad_rows((BC, LP)))
    pidx = pm.reshape(NW, NCH, CH, LP).transpose(0, 3, 1, 2).reshape(-1)
    clen = x_ctx_ent_len.reshape(-1)
    cm = jnp.where(jnp.arange(LC)[None, :] < clen[:, None],
                   x_ctx_ent.reshape(-1, LC), pad_rows((BC * NC, LC)))
    cidx = (cm.reshape(NW, NCH, CH, NC, LC)
            .transpose(0, 3, 4, 1, 2).reshape(-1))

    qinv = 1.0 / query_lengths.astype(f32)
    qinv_bc = jnp.broadcast_to(qinv[:, None], (B, C)).reshape(-1)
    wt = qinv_bc / tlen.astype(f32)
    wp = qinv_bc / plen.astype(f32)
    cnum = x_ctx_ent_num.reshape(-1)
    nmask = (jnp.arange(NC)[None, :] < cnum[:, None]).astype(f32)
    wc = (nmask * qinv_bc[:, None]
          / (x_ctx_ent_len.reshape(BC, NC).astype(f32)
             * cnum[:, None].astype(f32))).T.reshape(-1)
    amask = (jnp.arange(C)[None, :] < ans_num[:, None]).astype(f32).reshape(-1)

    D2 = D
    table_bf = emb_table.astype(jnp.bfloat16)
    table_p = jnp.concatenate(
        [table_bf, jnp.zeros((NPAD, D2), jnp.bfloat16)], axis=0)
    fn = _make_sc_kernel(B, C, NC, LT, LP, LC, LQ, V + NPAD, D, D2)
    score = fn(table_p, qidx, tidx, pidx, cidx, wt, wp, wc, amask)
    return score.reshape(B, C)
